# baseline (speedup 1.0000x reference)
import functools

import jax
import jax.numpy as jnp
from jax import lax
from jax.experimental import pallas as pl
from jax.experimental.pallas import tpu as pltpu
from jax.experimental.pallas import tpu_sc as plsc

N = 10000
E = 320000
D = 128
G = 512

NC = 2
NS = 16
NW = NC * NS

NPAD = 10240
ROWS_PT = NPAD // NS
RCH = 128
NRC = ROWS_PT // RCH

ECH = 128
EPT_CH = 80
GCH = 8
EPT = ECH * EPT_CH
EPAD = EPT * NW
POOL_PT = G // NS


def _sc_body(x_hbm, src_hbm, dst_hbm, ew_hbm, batch_hbm,
             agg_out, pool_out,
             agg_sh, pool_sh, src_all, dst_all, ew_all,
             rows0, bidx_v, sem0):
    c = lax.axis_index("c")
    s = lax.axis_index("s")
    wid = s * NC + c
    r0 = s * ROWS_PT

    zvec = jnp.zeros((16,), jnp.float32)

    def _zrow(i, carry):
        for j in range(D // 16):
            rows0[i, pl.ds(j * 16, 16)] = zvec
        return carry

    lax.fori_loop(0, ECH, _zrow, 0)

    @pl.when(c == 0)
    def _():
        for k in range(NRC):
            off = r0 + k * RCH
            pltpu.sync_copy(x_hbm.at[pl.ds(off, RCH)],
                            agg_sh.at[pl.ds(off, RCH)])

    @pl.when(c != 0)
    def _():
        for k in range(NRC):
            off = r0 + k * RCH
            pltpu.sync_copy(rows0.at[pl.ds(0, RCH)],
                            agg_sh.at[pl.ds(off, RCH)])

    pltpu.sync_copy(rows0.at[pl.ds(0, POOL_PT)],
                    pool_sh.at[pl.ds(s * POOL_PT, POOL_PT)])
    plsc.subcore_barrier()

    cbase = wid * EPT_CH

    def _scale(rows_v, ci):
        def body(g, inner):
            wv = ew_all[ci, pl.ds(g * 16, 16)]
            for e in range(16):
                w = wv[e]
                r = g * 16 + e
                for j in range(D // 16):
                    sl = pl.ds(j * 16, 16)
                    rows_v[r, sl] = rows_v[r, sl] * w
            return inner

        lax.fori_loop(0, ECH // 16, body, 0)

    def _egroup(gi, carry):
        goff = cbase + gi * GCH
        pltpu.sync_copy(src_hbm.at[pl.ds(goff, GCH)], src_all)
        pltpu.sync_copy(dst_hbm.at[pl.ds(goff, GCH)], dst_all)
        pltpu.sync_copy(ew_hbm.at[pl.ds(goff, GCH)], ew_all)

        def _echunk(a, inner):
            pltpu.async_copy(x_hbm.at[src_all.at[a]], rows0, sem0).wait()
            _scale(rows0, a)
            pltpu.sync_copy(rows0, agg_sh.at[dst_all.at[a]], add=True)
            return inner

        lax.fori_loop(0, GCH, _echunk, 0)
        return carry

    lax.fori_loop(0, EPT_CH // GCH, _egroup, 0)
    plsc.subcore_barrier()

    rfront = rows0.at[pl.ds(0, RCH)]
    for k in range(NRC):
        off = r0 + k * RCH
        pltpu.sync_copy(agg_sh.at[pl.ds(off, RCH)], rfront)
        pltpu.sync_copy(batch_hbm.at[pl.ds(off, RCH)], bidx_v)
        pltpu.sync_copy(rfront, agg_out.at[pl.ds(c * NPAD + off, RCH)])
        pltpu.sync_copy(rfront, pool_sh.at[bidx_v], add=True)
    plsc.subcore_barrier()

    pltpu.sync_copy(pool_sh.at[pl.ds(s * POOL_PT, POOL_PT)],
                    pool_out.at[pl.ds(c * G + s * POOL_PT, POOL_PT)])


_sc_call = pl.kernel(
    _sc_body,
    out_type=[
        jax.ShapeDtypeStruct((NC * NPAD, D), jnp.float32),
        jax.ShapeDtypeStruct((NC * G, D), jnp.float32),
    ],
    mesh=plsc.VectorSubcoreMesh(core_axis_name="c", subcore_axis_name="s"),
    scratch_types=[
        pltpu.VMEM_SHARED((NPAD, D), jnp.float32),
        pltpu.VMEM_SHARED((G, D), jnp.float32),
        pltpu.VMEM((GCH, ECH), jnp.int32),
        pltpu.VMEM((GCH, ECH), jnp.int32),
        pltpu.VMEM((GCH, ECH), jnp.float32),
        pltpu.VMEM((ECH, D), jnp.float32),
        pltpu.VMEM((RCH,), jnp.int32),
        pltpu.SemaphoreType.DMA,
    ],
)


def _small_body(p0_ref, p1_ref, w1_ref, w2_ref, wp_ref,
                g1_ref, gt_ref, wc_ref):
    s = p0_ref[...] + p1_ref[...]
    g1_ref[...] = jnp.dot(s, w1_ref[...], preferred_element_type=jnp.float32)
    gt_ref[...] = jnp.dot(s, w2_ref[...], preferred_element_type=jnp.float32)
    wc_ref[...] = jnp.dot(w2_ref[...], wp_ref[...],
                          preferred_element_type=jnp.float32)


def _pred_body(a0_ref, a1_ref, wc_ref, pv_ref, out_ref):
    a = a0_ref[...] + a1_ref[...]
    z = jnp.dot(a, wc_ref[...], preferred_element_type=jnp.float32)
    z = z + pv_ref[0:1, :]
    mu = jnp.mean(z, axis=-1, keepdims=True)
    zc = z - mu
    var = jnp.mean(zc * zc, axis=-1, keepdims=True)
    zn = zc * lax.rsqrt(var + 1e-5) * pv_ref[1:2, :] + pv_ref[2:3, :]
    alpha = pv_ref[3, 0]
    out_ref[...] = jnp.where(zn >= 0, zn, alpha * zn)


def kernel(x, edge_index, edge_weight, batch, W1, W2, Wp, bp, gamma, beta, alpha):
    x_pad = jnp.pad(x, ((0, NPAD - N), (0, 0)))
    batch_pad = jnp.pad(batch, (0, NPAD - N))
    src = jnp.pad(edge_index[0], (0, EPAD - E)).reshape(NW * EPT_CH, ECH)
    dst = jnp.pad(edge_index[1], (0, EPAD - E)).reshape(NW * EPT_CH, ECH)
    ew = jnp.pad(edge_weight, (0, EPAD - E)).reshape(NW * EPT_CH, ECH)

    agg_parts, pool_parts = _sc_call(x_pad, src, dst, ew, batch_pad)

    g1, gt, wc = pl.pallas_call(
        _small_body,
        out_shape=[
            jax.ShapeDtypeStruct((G, D), jnp.float32),
            jax.ShapeDtypeStruct((G, D), jnp.float32),
            jax.ShapeDtypeStruct((D, D), jnp.float32),
        ],
    )(pool_parts[:G], pool_parts[G:], W1, W2, Wp)

    pvec = jnp.stack([bp, gamma, beta,
                      jnp.full((D,), alpha, dtype=jnp.float32)] + [bp] * 4)

    nb = 8
    blk = NPAD // nb
    h_full = pl.pallas_call(
        _pred_body,
        grid=(nb,),
        in_specs=[
            pl.BlockSpec((blk, D), lambda i: (i, 0)),
            pl.BlockSpec((blk, D), lambda i: (i, 0)),
            pl.BlockSpec((D, D), lambda i: (0, 0)),
            pl.BlockSpec((8, D), lambda i: (0, 0)),
        ],
        out_specs=pl.BlockSpec((blk, D), lambda i: (i, 0)),
        out_shape=jax.ShapeDtypeStruct((NPAD, D), jnp.float32),
    )(agg_parts[:NPAD], agg_parts[NPAD:], wc, pvec)

    h_pred = h_full[:N]
    return (g1, g1, h_pred, h_pred, gt, gt)


# speedup vs baseline: 1.2408x; 1.2408x over previous
import functools

import jax
import jax.numpy as jnp
from jax import lax
from jax.experimental import pallas as pl
from jax.experimental.pallas import tpu as pltpu
from jax.experimental.pallas import tpu_sc as plsc

N = 10000
E = 320000
D = 128
G = 512

NC = 2
NS = 16
NW = NC * NS

NPAD = 10240
ROWS_PT = NPAD // NS
RCH = 128
NRC = ROWS_PT // RCH

ECH = 128
EPT_CH = 80
GCH = 8
EPT = ECH * EPT_CH
EPAD = EPT * NW
POOL_PT = G // NS


def _sc_body(x_hbm, src_hbm, dst_hbm, ew_hbm, batch_hbm,
             agg_out, pool_out,
             agg_sh, pool_sh, src0, dst0, ew0, src1, dst1, ew1,
             rows0, rows1, bidx_v, semg0, semg1, semi0, semi1):
    c = lax.axis_index("c")
    s = lax.axis_index("s")
    wid = s * NC + c
    r0 = s * ROWS_PT

    zvec = jnp.zeros((16,), jnp.float32)

    def _zrow(i, carry):
        for j in range(D // 16):
            rows0[i, pl.ds(j * 16, 16)] = zvec
        return carry

    lax.fori_loop(0, ECH, _zrow, 0)

    @pl.when(c == 0)
    def _():
        for k in range(NRC):
            off = r0 + k * RCH
            pltpu.sync_copy(x_hbm.at[pl.ds(off, RCH)],
                            agg_sh.at[pl.ds(off, RCH)])

    @pl.when(c != 0)
    def _():
        for k in range(NRC):
            off = r0 + k * RCH
            pltpu.sync_copy(rows0.at[pl.ds(0, RCH)],
                            agg_sh.at[pl.ds(off, RCH)])

    pltpu.sync_copy(rows0.at[pl.ds(0, POOL_PT)],
                    pool_sh.at[pl.ds(s * POOL_PT, POOL_PT)])
    plsc.subcore_barrier()

    ebase = wid * EPT

    def _scale(rows_v, ewv):
        def body(g, inner):
            wv = ewv[pl.ds(g * 16, 16)]
            for e in range(16):
                w = wv[e]
                r = g * 16 + e
                for j in range(D // 16):
                    sl = pl.ds(j * 16, 16)
                    rows_v[r, sl] = rows_v[r, sl] * w
            return inner

        lax.fori_loop(0, ECH // 16, body, 0)

    def _idx_issue(ci, srcv, dstv, ewv, sem):
        e = ebase + ci * ECH
        pltpu.async_copy(src_hbm.at[pl.ds(e, ECH)], srcv, sem)
        pltpu.async_copy(dst_hbm.at[pl.ds(e, ECH)], dstv, sem)
        pltpu.async_copy(ew_hbm.at[pl.ds(e, ECH)], ewv, sem)

    def _idx_wait(srcv, dstv, ewv, sem):
        z = pl.ds(0, ECH)
        pltpu.make_async_copy(src_hbm.at[z], srcv, sem).wait()
        pltpu.make_async_copy(dst_hbm.at[z], dstv, sem).wait()
        pltpu.make_async_copy(ew_hbm.at[z], ewv, sem).wait()

    _idx_issue(0, src0, dst0, ew0, semi0)
    _idx_wait(src0, dst0, ew0, semi0)
    pltpu.async_copy(x_hbm.at[src0], rows0, semg0)
    _idx_issue(1, src1, dst1, ew1, semi1)

    NIT = EPT_CH // 2

    def _epair(it, carry):
        a = 2 * it
        pltpu.make_async_copy(x_hbm.at[src0], rows0, semg0).wait()
        _idx_wait(src1, dst1, ew1, semi1)
        pltpu.async_copy(x_hbm.at[src1], rows1, semg1)
        _scale(rows0, ew0)
        pltpu.sync_copy(rows0, agg_sh.at[dst0], add=True)

        @pl.when(it < NIT - 1)
        def _():
            _idx_issue(a + 2, src0, dst0, ew0, semi0)

        pltpu.make_async_copy(x_hbm.at[src1], rows1, semg1).wait()

        @pl.when(it < NIT - 1)
        def _():
            _idx_wait(src0, dst0, ew0, semi0)
            pltpu.async_copy(x_hbm.at[src0], rows0, semg0)

        _scale(rows1, ew1)
        pltpu.sync_copy(rows1, agg_sh.at[dst1], add=True)

        @pl.when(it < NIT - 1)
        def _():
            _idx_issue(a + 3, src1, dst1, ew1, semi1)

        return carry

    lax.fori_loop(0, NIT, _epair, 0)
    plsc.subcore_barrier()

    rfront = rows0.at[pl.ds(0, RCH)]
    for k in range(NRC):
        off = r0 + k * RCH
        pltpu.sync_copy(agg_sh.at[pl.ds(off, RCH)], rfront)
        pltpu.sync_copy(batch_hbm.at[pl.ds(off, RCH)], bidx_v)
        pltpu.sync_copy(rfront, agg_out.at[pl.ds(c * NPAD + off, RCH)])
        pltpu.sync_copy(rfront, pool_sh.at[bidx_v], add=True)
    plsc.subcore_barrier()

    pltpu.sync_copy(pool_sh.at[pl.ds(s * POOL_PT, POOL_PT)],
                    pool_out.at[pl.ds(c * G + s * POOL_PT, POOL_PT)])


_sc_call = pl.kernel(
    _sc_body,
    out_type=[
        jax.ShapeDtypeStruct((NC * NPAD, D), jnp.float32),
        jax.ShapeDtypeStruct((NC * G, D), jnp.float32),
    ],
    mesh=plsc.VectorSubcoreMesh(core_axis_name="c", subcore_axis_name="s"),
    scratch_types=[
        pltpu.VMEM_SHARED((NPAD, D), jnp.float32),
        pltpu.VMEM_SHARED((G, D), jnp.float32),
        pltpu.VMEM((ECH,), jnp.int32),
        pltpu.VMEM((ECH,), jnp.int32),
        pltpu.VMEM((ECH,), jnp.float32),
        pltpu.VMEM((ECH,), jnp.int32),
        pltpu.VMEM((ECH,), jnp.int32),
        pltpu.VMEM((ECH,), jnp.float32),
        pltpu.VMEM((ECH, D), jnp.float32),
        pltpu.VMEM((ECH, D), jnp.float32),
        pltpu.VMEM((RCH,), jnp.int32),
        pltpu.SemaphoreType.DMA,
        pltpu.SemaphoreType.DMA,
        pltpu.SemaphoreType.DMA,
        pltpu.SemaphoreType.DMA,
    ],
)


def _small_body(p0_ref, p1_ref, w1_ref, w2_ref, wp_ref,
                g1_ref, gt_ref, wc_ref):
    s = p0_ref[...] + p1_ref[...]
    g1_ref[...] = jnp.dot(s, w1_ref[...], preferred_element_type=jnp.float32)
    gt_ref[...] = jnp.dot(s, w2_ref[...], preferred_element_type=jnp.float32)
    wc_ref[...] = jnp.dot(w2_ref[...], wp_ref[...],
                          preferred_element_type=jnp.float32)


def _pred_body(a0_ref, a1_ref, wc_ref, pv_ref, out_ref):
    a = a0_ref[...] + a1_ref[...]
    z = jnp.dot(a, wc_ref[...], preferred_element_type=jnp.float32)
    z = z + pv_ref[0:1, :]
    mu = jnp.mean(z, axis=-1, keepdims=True)
    zc = z - mu
    var = jnp.mean(zc * zc, axis=-1, keepdims=True)
    zn = zc * lax.rsqrt(var + 1e-5) * pv_ref[1:2, :] + pv_ref[2:3, :]
    alpha = pv_ref[3, 0]
    out_ref[...] = jnp.where(zn >= 0, zn, alpha * zn)


def kernel(x, edge_index, edge_weight, batch, W1, W2, Wp, bp, gamma, beta, alpha):
    x_pad = jnp.pad(x, ((0, NPAD - N), (0, 0)))
    batch_pad = jnp.pad(batch, (0, NPAD - N))
    src = jnp.pad(edge_index[0], (0, EPAD - E))
    dst = jnp.pad(edge_index[1], (0, EPAD - E))
    ew = jnp.pad(edge_weight, (0, EPAD - E))

    agg_parts, pool_parts = _sc_call(x_pad, src, dst, ew, batch_pad)

    g1, gt, wc = pl.pallas_call(
        _small_body,
        out_shape=[
            jax.ShapeDtypeStruct((G, D), jnp.float32),
            jax.ShapeDtypeStruct((G, D), jnp.float32),
            jax.ShapeDtypeStruct((D, D), jnp.float32),
        ],
    )(pool_parts[:G], pool_parts[G:], W1, W2, Wp)

    pvec = jnp.stack([bp, gamma, beta,
                      jnp.full((D,), alpha, dtype=jnp.float32)] + [bp] * 4)

    nb = 8
    blk = NPAD // nb
    h_full = pl.pallas_call(
        _pred_body,
        grid=(nb,),
        in_specs=[
            pl.BlockSpec((blk, D), lambda i: (i, 0)),
            pl.BlockSpec((blk, D), lambda i: (i, 0)),
            pl.BlockSpec((D, D), lambda i: (0, 0)),
            pl.BlockSpec((8, D), lambda i: (0, 0)),
        ],
        out_specs=pl.BlockSpec((blk, D), lambda i: (i, 0)),
        out_shape=jax.ShapeDtypeStruct((NPAD, D), jnp.float32),
    )(agg_parts[:NPAD], agg_parts[NPAD:], wc, pvec)

    h_pred = h_full[:N]
    return (g1, g1, h_pred, h_pred, gt, gt)


# speedup vs baseline: 1.2585x; 1.0142x over previous
import functools

import jax
import jax.numpy as jnp
from jax import lax
from jax.experimental import pallas as pl
from jax.experimental.pallas import tpu as pltpu
from jax.experimental.pallas import tpu_sc as plsc

N = 10000
E = 320000
D = 128
G = 512

NC = 2
NS = 16
NW = NC * NS

_DIAG = "noscale"

NPAD = 10240
ROWS_PT = NPAD // NS
RCH = 128
NRC = ROWS_PT // RCH

ECH = 128
EPT_CH = 80
GCH = 8
EPT = ECH * EPT_CH
EPAD = EPT * NW
POOL_PT = G // NS


def _sc_body(x_hbm, src_hbm, dst_hbm, ew_hbm, batch_hbm,
             agg_out, pool_out,
             agg_sh, pool_sh, src0, dst0, ew0, src1, dst1, ew1,
             rows0, rows1, bidx_v, semg0, semg1, semi0, semi1):
    c = lax.axis_index("c")
    s = lax.axis_index("s")
    wid = s * NC + c
    r0 = s * ROWS_PT

    zvec = jnp.zeros((16,), jnp.float32)

    def _zrow(i, carry):
        for j in range(D // 16):
            rows0[i, pl.ds(j * 16, 16)] = zvec
        return carry

    lax.fori_loop(0, ECH, _zrow, 0)

    @pl.when(c == 0)
    def _():
        for k in range(NRC):
            off = r0 + k * RCH
            pltpu.sync_copy(x_hbm.at[pl.ds(off, RCH)],
                            agg_sh.at[pl.ds(off, RCH)])

    @pl.when(c != 0)
    def _():
        for k in range(NRC):
            off = r0 + k * RCH
            pltpu.sync_copy(rows0.at[pl.ds(0, RCH)],
                            agg_sh.at[pl.ds(off, RCH)])

    pltpu.sync_copy(rows0.at[pl.ds(0, POOL_PT)],
                    pool_sh.at[pl.ds(s * POOL_PT, POOL_PT)])
    plsc.subcore_barrier()

    ebase = wid * EPT

    def _scale(rows_v, ewv):
        def body(g, inner):
            wv = ewv[pl.ds(g * 16, 16)]
            for e in range(16):
                w = wv[e]
                r = g * 16 + e
                for j in range(D // 16):
                    sl = pl.ds(j * 16, 16)
                    rows_v[r, sl] = rows_v[r, sl] * w
            return inner

        lax.fori_loop(0, ECH // 16, body, 0)

    def _idx_issue(ci, srcv, dstv, ewv, sem):
        e = ebase + ci * ECH
        pltpu.async_copy(src_hbm.at[pl.ds(e, ECH)], srcv, sem)
        pltpu.async_copy(dst_hbm.at[pl.ds(e, ECH)], dstv, sem)
        pltpu.async_copy(ew_hbm.at[pl.ds(e, ECH)], ewv, sem)

    def _idx_wait(srcv, dstv, ewv, sem):
        z = pl.ds(0, ECH)
        pltpu.make_async_copy(src_hbm.at[z], srcv, sem).wait()
        pltpu.make_async_copy(dst_hbm.at[z], dstv, sem).wait()
        pltpu.make_async_copy(ew_hbm.at[z], ewv, sem).wait()

    _idx_issue(0, src0, dst0, ew0, semi0)
    _idx_wait(src0, dst0, ew0, semi0)
    pltpu.async_copy(x_hbm.at[src0], rows0, semg0)
    _idx_issue(1, src1, dst1, ew1, semi1)

    NIT = EPT_CH // 2

    def _epair(it, carry):
        a = 2 * it
        pltpu.make_async_copy(x_hbm.at[src0], rows0, semg0).wait()
        _idx_wait(src1, dst1, ew1, semi1)
        pltpu.async_copy(x_hbm.at[src1], rows1, semg1)
        if _DIAG != "noscale":
            _scale(rows0, ew0)
        if _DIAG != "noscatter":
            pltpu.sync_copy(rows0, agg_sh.at[dst0], add=True)

        @pl.when(it < NIT - 1)
        def _():
            _idx_issue(a + 2, src0, dst0, ew0, semi0)

        pltpu.make_async_copy(x_hbm.at[src1], rows1, semg1).wait()

        @pl.when(it < NIT - 1)
        def _():
            _idx_wait(src0, dst0, ew0, semi0)
            pltpu.async_copy(x_hbm.at[src0], rows0, semg0)

        if _DIAG != "noscale":
            _scale(rows1, ew1)
        if _DIAG != "noscatter":
            pltpu.sync_copy(rows1, agg_sh.at[dst1], add=True)

        @pl.when(it < NIT - 1)
        def _():
            _idx_issue(a + 3, src1, dst1, ew1, semi1)

        return carry

    lax.fori_loop(0, NIT, _epair, 0)
    plsc.subcore_barrier()

    rfront = rows0.at[pl.ds(0, RCH)]
    for k in range(NRC):
        off = r0 + k * RCH
        pltpu.sync_copy(agg_sh.at[pl.ds(off, RCH)], rfront)
        pltpu.sync_copy(batch_hbm.at[pl.ds(off, RCH)], bidx_v)
        pltpu.sync_copy(rfront, agg_out.at[pl.ds(c * NPAD + off, RCH)])
        pltpu.sync_copy(rfront, pool_sh.at[bidx_v], add=True)
    plsc.subcore_barrier()

    pltpu.sync_copy(pool_sh.at[pl.ds(s * POOL_PT, POOL_PT)],
                    pool_out.at[pl.ds(c * G + s * POOL_PT, POOL_PT)])


_sc_call = pl.kernel(
    _sc_body,
    out_type=[
        jax.ShapeDtypeStruct((NC * NPAD, D), jnp.float32),
        jax.ShapeDtypeStruct((NC * G, D), jnp.float32),
    ],
    mesh=plsc.VectorSubcoreMesh(core_axis_name="c", subcore_axis_name="s"),
    scratch_types=[
        pltpu.VMEM_SHARED((NPAD, D), jnp.float32),
        pltpu.VMEM_SHARED((G, D), jnp.float32),
        pltpu.VMEM((ECH,), jnp.int32),
        pltpu.VMEM((ECH,), jnp.int32),
        pltpu.VMEM((ECH,), jnp.float32),
        pltpu.VMEM((ECH,), jnp.int32),
        pltpu.VMEM((ECH,), jnp.int32),
        pltpu.VMEM((ECH,), jnp.float32),
        pltpu.VMEM((ECH, D), jnp.float32),
        pltpu.VMEM((ECH, D), jnp.float32),
        pltpu.VMEM((RCH,), jnp.int32),
        pltpu.SemaphoreType.DMA,
        pltpu.SemaphoreType.DMA,
        pltpu.SemaphoreType.DMA,
        pltpu.SemaphoreType.DMA,
    ],
)


def _small_body(p0_ref, p1_ref, w1_ref, w2_ref, wp_ref,
                g1_ref, gt_ref, wc_ref):
    s = p0_ref[...] + p1_ref[...]
    g1_ref[...] = jnp.dot(s, w1_ref[...], preferred_element_type=jnp.float32)
    gt_ref[...] = jnp.dot(s, w2_ref[...], preferred_element_type=jnp.float32)
    wc_ref[...] = jnp.dot(w2_ref[...], wp_ref[...],
                          preferred_element_type=jnp.float32)


def _pred_body(a0_ref, a1_ref, wc_ref, pv_ref, out_ref):
    a = a0_ref[...] + a1_ref[...]
    z = jnp.dot(a, wc_ref[...], preferred_element_type=jnp.float32)
    z = z + pv_ref[0:1, :]
    mu = jnp.mean(z, axis=-1, keepdims=True)
    zc = z - mu
    var = jnp.mean(zc * zc, axis=-1, keepdims=True)
    zn = zc * lax.rsqrt(var + 1e-5) * pv_ref[1:2, :] + pv_ref[2:3, :]
    alpha = pv_ref[3, 0]
    out_ref[...] = jnp.where(zn >= 0, zn, alpha * zn)


def kernel(x, edge_index, edge_weight, batch, W1, W2, Wp, bp, gamma, beta, alpha):
    x_pad = jnp.pad(x, ((0, NPAD - N), (0, 0)))
    batch_pad = jnp.pad(batch, (0, NPAD - N))
    src = jnp.pad(edge_index[0], (0, EPAD - E))
    dst = jnp.pad(edge_index[1], (0, EPAD - E))
    ew = jnp.pad(edge_weight, (0, EPAD - E))

    agg_parts, pool_parts = _sc_call(x_pad, src, dst, ew, batch_pad)

    g1, gt, wc = pl.pallas_call(
        _small_body,
        out_shape=[
            jax.ShapeDtypeStruct((G, D), jnp.float32),
            jax.ShapeDtypeStruct((G, D), jnp.float32),
            jax.ShapeDtypeStruct((D, D), jnp.float32),
        ],
    )(pool_parts[:G], pool_parts[G:], W1, W2, Wp)

    pvec = jnp.stack([bp, gamma, beta,
                      jnp.full((D,), alpha, dtype=jnp.float32)] + [bp] * 4)

    nb = 8
    blk = NPAD // nb
    h_full = pl.pallas_call(
        _pred_body,
        grid=(nb,),
        in_specs=[
            pl.BlockSpec((blk, D), lambda i: (i, 0)),
            pl.BlockSpec((blk, D), lambda i: (i, 0)),
            pl.BlockSpec((D, D), lambda i: (0, 0)),
            pl.BlockSpec((8, D), lambda i: (0, 0)),
        ],
        out_specs=pl.BlockSpec((blk, D), lambda i: (i, 0)),
        out_shape=jax.ShapeDtypeStruct((NPAD, D), jnp.float32),
    )(agg_parts[:NPAD], agg_parts[NPAD:], wc, pvec)

    h_pred = h_full[:N]
    return (g1, g1, h_pred, h_pred, gt, gt)


# speedup vs baseline: 1.2609x; 1.0019x over previous
import functools

import jax
import jax.numpy as jnp
from jax import lax
from jax.experimental import pallas as pl
from jax.experimental.pallas import tpu as pltpu
from jax.experimental.pallas import tpu_sc as plsc

N = 10000
E = 320000
D = 128
G = 512

NC = 2
NS = 16
NW = NC * NS

_DIAG = "noscatter"

NPAD = 10240
ROWS_PT = NPAD // NS
RCH = 128
NRC = ROWS_PT // RCH

ECH = 128
EPT_CH = 80
GCH = 8
EPT = ECH * EPT_CH
EPAD = EPT * NW
POOL_PT = G // NS


def _sc_body(x_hbm, src_hbm, dst_hbm, ew_hbm, batch_hbm,
             agg_out, pool_out,
             agg_sh, pool_sh, src0, dst0, ew0, src1, dst1, ew1,
             rows0, rows1, bidx_v, semg0, semg1, semi0, semi1):
    c = lax.axis_index("c")
    s = lax.axis_index("s")
    wid = s * NC + c
    r0 = s * ROWS_PT

    zvec = jnp.zeros((16,), jnp.float32)

    def _zrow(i, carry):
        for j in range(D // 16):
            rows0[i, pl.ds(j * 16, 16)] = zvec
        return carry

    lax.fori_loop(0, ECH, _zrow, 0)

    @pl.when(c == 0)
    def _():
        for k in range(NRC):
            off = r0 + k * RCH
            pltpu.sync_copy(x_hbm.at[pl.ds(off, RCH)],
                            agg_sh.at[pl.ds(off, RCH)])

    @pl.when(c != 0)
    def _():
        for k in range(NRC):
            off = r0 + k * RCH
            pltpu.sync_copy(rows0.at[pl.ds(0, RCH)],
                            agg_sh.at[pl.ds(off, RCH)])

    pltpu.sync_copy(rows0.at[pl.ds(0, POOL_PT)],
                    pool_sh.at[pl.ds(s * POOL_PT, POOL_PT)])
    plsc.subcore_barrier()

    ebase = wid * EPT

    def _scale(rows_v, ewv):
        def body(g, inner):
            wv = ewv[pl.ds(g * 16, 16)]
            for e in range(16):
                w = wv[e]
                r = g * 16 + e
                for j in range(D // 16):
                    sl = pl.ds(j * 16, 16)
                    rows_v[r, sl] = rows_v[r, sl] * w
            return inner

        lax.fori_loop(0, ECH // 16, body, 0)

    def _idx_issue(ci, srcv, dstv, ewv, sem):
        e = ebase + ci * ECH
        pltpu.async_copy(src_hbm.at[pl.ds(e, ECH)], srcv, sem)
        pltpu.async_copy(dst_hbm.at[pl.ds(e, ECH)], dstv, sem)
        pltpu.async_copy(ew_hbm.at[pl.ds(e, ECH)], ewv, sem)

    def _idx_wait(srcv, dstv, ewv, sem):
        z = pl.ds(0, ECH)
        pltpu.make_async_copy(src_hbm.at[z], srcv, sem).wait()
        pltpu.make_async_copy(dst_hbm.at[z], dstv, sem).wait()
        pltpu.make_async_copy(ew_hbm.at[z], ewv, sem).wait()

    _idx_issue(0, src0, dst0, ew0, semi0)
    _idx_wait(src0, dst0, ew0, semi0)
    pltpu.async_copy(x_hbm.at[src0], rows0, semg0)
    _idx_issue(1, src1, dst1, ew1, semi1)

    NIT = EPT_CH // 2

    def _epair(it, carry):
        a = 2 * it
        pltpu.make_async_copy(x_hbm.at[src0], rows0, semg0).wait()
        _idx_wait(src1, dst1, ew1, semi1)
        pltpu.async_copy(x_hbm.at[src1], rows1, semg1)
        if _DIAG != "noscale":
            _scale(rows0, ew0)
        if _DIAG != "noscatter":
            pltpu.sync_copy(rows0, agg_sh.at[dst0], add=True)

        @pl.when(it < NIT - 1)
        def _():
            _idx_issue(a + 2, src0, dst0, ew0, semi0)

        pltpu.make_async_copy(x_hbm.at[src1], rows1, semg1).wait()

        @pl.when(it < NIT - 1)
        def _():
            _idx_wait(src0, dst0, ew0, semi0)
            pltpu.async_copy(x_hbm.at[src0], rows0, semg0)

        if _DIAG != "noscale":
            _scale(rows1, ew1)
        if _DIAG != "noscatter":
            pltpu.sync_copy(rows1, agg_sh.at[dst1], add=True)

        @pl.when(it < NIT - 1)
        def _():
            _idx_issue(a + 3, src1, dst1, ew1, semi1)

        return carry

    lax.fori_loop(0, NIT, _epair, 0)
    plsc.subcore_barrier()

    rfront = rows0.at[pl.ds(0, RCH)]
    for k in range(NRC):
        off = r0 + k * RCH
        pltpu.sync_copy(agg_sh.at[pl.ds(off, RCH)], rfront)
        pltpu.sync_copy(batch_hbm.at[pl.ds(off, RCH)], bidx_v)
        pltpu.sync_copy(rfront, agg_out.at[pl.ds(c * NPAD + off, RCH)])
        pltpu.sync_copy(rfront, pool_sh.at[bidx_v], add=True)
    plsc.subcore_barrier()

    pltpu.sync_copy(pool_sh.at[pl.ds(s * POOL_PT, POOL_PT)],
                    pool_out.at[pl.ds(c * G + s * POOL_PT, POOL_PT)])


_sc_call = pl.kernel(
    _sc_body,
    out_type=[
        jax.ShapeDtypeStruct((NC * NPAD, D), jnp.float32),
        jax.ShapeDtypeStruct((NC * G, D), jnp.float32),
    ],
    mesh=plsc.VectorSubcoreMesh(core_axis_name="c", subcore_axis_name="s"),
    scratch_types=[
        pltpu.VMEM_SHARED((NPAD, D), jnp.float32),
        pltpu.VMEM_SHARED((G, D), jnp.float32),
        pltpu.VMEM((ECH,), jnp.int32),
        pltpu.VMEM((ECH,), jnp.int32),
        pltpu.VMEM((ECH,), jnp.float32),
        pltpu.VMEM((ECH,), jnp.int32),
        pltpu.VMEM((ECH,), jnp.int32),
        pltpu.VMEM((ECH,), jnp.float32),
        pltpu.VMEM((ECH, D), jnp.float32),
        pltpu.VMEM((ECH, D), jnp.float32),
        pltpu.VMEM((RCH,), jnp.int32),
        pltpu.SemaphoreType.DMA,
        pltpu.SemaphoreType.DMA,
        pltpu.SemaphoreType.DMA,
        pltpu.SemaphoreType.DMA,
    ],
)


def _small_body(p0_ref, p1_ref, w1_ref, w2_ref, wp_ref,
                g1_ref, gt_ref, wc_ref):
    s = p0_ref[...] + p1_ref[...]
    g1_ref[...] = jnp.dot(s, w1_ref[...], preferred_element_type=jnp.float32)
    gt_ref[...] = jnp.dot(s, w2_ref[...], preferred_element_type=jnp.float32)
    wc_ref[...] = jnp.dot(w2_ref[...], wp_ref[...],
                          preferred_element_type=jnp.float32)


def _pred_body(a0_ref, a1_ref, wc_ref, pv_ref, out_ref):
    a = a0_ref[...] + a1_ref[...]
    z = jnp.dot(a, wc_ref[...], preferred_element_type=jnp.float32)
    z = z + pv_ref[0:1, :]
    mu = jnp.mean(z, axis=-1, keepdims=True)
    zc = z - mu
    var = jnp.mean(zc * zc, axis=-1, keepdims=True)
    zn = zc * lax.rsqrt(var + 1e-5) * pv_ref[1:2, :] + pv_ref[2:3, :]
    alpha = pv_ref[3, 0]
    out_ref[...] = jnp.where(zn >= 0, zn, alpha * zn)


def kernel(x, edge_index, edge_weight, batch, W1, W2, Wp, bp, gamma, beta, alpha):
    x_pad = jnp.pad(x, ((0, NPAD - N), (0, 0)))
    batch_pad = jnp.pad(batch, (0, NPAD - N))
    src = jnp.pad(edge_index[0], (0, EPAD - E))
    dst = jnp.pad(edge_index[1], (0, EPAD - E))
    ew = jnp.pad(edge_weight, (0, EPAD - E))

    agg_parts, pool_parts = _sc_call(x_pad, src, dst, ew, batch_pad)

    g1, gt, wc = pl.pallas_call(
        _small_body,
        out_shape=[
            jax.ShapeDtypeStruct((G, D), jnp.float32),
            jax.ShapeDtypeStruct((G, D), jnp.float32),
            jax.ShapeDtypeStruct((D, D), jnp.float32),
        ],
    )(pool_parts[:G], pool_parts[G:], W1, W2, Wp)

    pvec = jnp.stack([bp, gamma, beta,
                      jnp.full((D,), alpha, dtype=jnp.float32)] + [bp] * 4)

    nb = 8
    blk = NPAD // nb
    h_full = pl.pallas_call(
        _pred_body,
        grid=(nb,),
        in_specs=[
            pl.BlockSpec((blk, D), lambda i: (i, 0)),
            pl.BlockSpec((blk, D), lambda i: (i, 0)),
            pl.BlockSpec((D, D), lambda i: (0, 0)),
            pl.BlockSpec((8, D), lambda i: (0, 0)),
        ],
        out_specs=pl.BlockSpec((blk, D), lambda i: (i, 0)),
        out_shape=jax.ShapeDtypeStruct((NPAD, D), jnp.float32),
    )(agg_parts[:NPAD], agg_parts[NPAD:], wc, pvec)

    h_pred = h_full[:N]
    return (g1, g1, h_pred, h_pred, gt, gt)


# speedup vs baseline: 1.2837x; 1.0181x over previous
import functools

import jax
import jax.numpy as jnp
from jax import lax
from jax.experimental import pallas as pl
from jax.experimental.pallas import tpu as pltpu
from jax.experimental.pallas import tpu_sc as plsc

N = 10000
E = 320000
D = 128
G = 512

NC = 2
NS = 16
NW = NC * NS

NPAD = 10240
ROWS_PT = NPAD // NS
RCH = 64
NRC = ROWS_PT // RCH

ECH = 64
NBUF = 4
EPT_CH = 160
EPT = ECH * EPT_CH
EPAD = EPT * NW
POOL_PT = G // NS


def _sc_body(x_hbm, src_hbm, dst_hbm, ew_hbm, batch_hbm,
             agg_out, pool_out,
             agg_sh, pool_sh,
             src0, dst0, ew0, src1, dst1, ew1,
             src2, dst2, ew2, src3, dst3, ew3,
             rows0, rows1, rows2, rows3, zbuf, bidx_v,
             semg0, semg1, semg2, semg3, semi0, semi1, semi2, semi3):
    c = lax.axis_index("c")
    s = lax.axis_index("s")
    wid = s * NC + c
    r0 = s * ROWS_PT
    srcs = (src0, src1, src2, src3)
    dsts = (dst0, dst1, dst2, dst3)
    ews = (ew0, ew1, ew2, ew3)
    rows = (rows0, rows1, rows2, rows3)
    semgs = (semg0, semg1, semg2, semg3)
    semis = (semi0, semi1, semi2, semi3)

    zvec = jnp.zeros((16,), jnp.float32)

    def _zrow(i, carry):
        for j in range(D // 16):
            zbuf[i, pl.ds(j * 16, 16)] = zvec
        return carry

    lax.fori_loop(0, RCH, _zrow, 0)

    @pl.when(c == 0)
    def _():
        for k in range(NRC):
            off = r0 + k * RCH
            pltpu.sync_copy(x_hbm.at[pl.ds(off, RCH)],
                            agg_sh.at[pl.ds(off, RCH)])

    @pl.when(c != 0)
    def _():
        for k in range(NRC):
            off = r0 + k * RCH
            pltpu.sync_copy(zbuf, agg_sh.at[pl.ds(off, RCH)])

    pltpu.sync_copy(zbuf.at[pl.ds(0, POOL_PT)],
                    pool_sh.at[pl.ds(s * POOL_PT, POOL_PT)])
    plsc.subcore_barrier()

    ebase = wid * EPT

    def _scale(rows_v, ewv):
        def body(g, inner):
            wv = ewv[pl.ds(g * 16, 16)]
            for e in range(16):
                w = wv[e]
                r = g * 16 + e
                for j in range(D // 16):
                    sl = pl.ds(j * 16, 16)
                    rows_v[r, sl] = rows_v[r, sl] * w
            return inner

        lax.fori_loop(0, ECH // 16, body, 0)

    def _idx_issue(ci, q):
        e = ebase + ci * ECH
        pltpu.async_copy(src_hbm.at[pl.ds(e, ECH)], srcs[q], semis[q])
        pltpu.async_copy(dst_hbm.at[pl.ds(e, ECH)], dsts[q], semis[q])
        pltpu.async_copy(ew_hbm.at[pl.ds(e, ECH)], ews[q], semis[q])

    def _idx_wait(q):
        z = pl.ds(0, ECH)
        pltpu.make_async_copy(src_hbm.at[z], srcs[q], semis[q]).wait()
        pltpu.make_async_copy(dst_hbm.at[z], dsts[q], semis[q]).wait()
        pltpu.make_async_copy(ew_hbm.at[z], ews[q], semis[q]).wait()

    for q in range(NBUF):
        _idx_issue(q, q)
    for q in range(NBUF):
        _idx_wait(q)
        pltpu.async_copy(x_hbm.at[srcs[q]], rows[q], semgs[q])

    NIT = EPT_CH // NBUF

    def _eround(it, carry):
        for q in range(NBUF):
            ci = NBUF * it + q
            pltpu.make_async_copy(x_hbm.at[srcs[q]], rows[q],
                                  semgs[q]).wait()
            _scale(rows[q], ews[q])
            pltpu.sync_copy(rows[q], agg_sh.at[dsts[q]], add=True)

            @pl.when(ci < EPT_CH - NBUF)
            def _():
                _idx_issue(ci + NBUF, q)
                _idx_wait(q)
                pltpu.async_copy(x_hbm.at[srcs[q]], rows[q], semgs[q])

        return carry

    lax.fori_loop(0, NIT, _eround, 0)
    plsc.subcore_barrier()

    rfront = zbuf
    for k in range(NRC):
        off = r0 + k * RCH
        pltpu.sync_copy(agg_sh.at[pl.ds(off, RCH)], rfront)
        pltpu.sync_copy(batch_hbm.at[pl.ds(off, RCH)], bidx_v)
        pltpu.sync_copy(rfront, agg_out.at[pl.ds(c * NPAD + off, RCH)])
        pltpu.sync_copy(rfront, pool_sh.at[bidx_v], add=True)
    plsc.subcore_barrier()

    pltpu.sync_copy(pool_sh.at[pl.ds(s * POOL_PT, POOL_PT)],
                    pool_out.at[pl.ds(c * G + s * POOL_PT, POOL_PT)])


_sc_call = pl.kernel(
    _sc_body,
    out_type=[
        jax.ShapeDtypeStruct((NC * NPAD, D), jnp.float32),
        jax.ShapeDtypeStruct((NC * G, D), jnp.float32),
    ],
    mesh=plsc.VectorSubcoreMesh(core_axis_name="c", subcore_axis_name="s"),
    scratch_types=[
        pltpu.VMEM_SHARED((NPAD, D), jnp.float32),
        pltpu.VMEM_SHARED((G, D), jnp.float32),
        *[t for _ in range(NBUF)
          for t in (pltpu.VMEM((ECH,), jnp.int32),
                    pltpu.VMEM((ECH,), jnp.int32),
                    pltpu.VMEM((ECH,), jnp.float32))],
        *[pltpu.VMEM((ECH, D), jnp.float32) for _ in range(NBUF)],
        pltpu.VMEM((RCH, D), jnp.float32),
        pltpu.VMEM((RCH,), jnp.int32),
        *[pltpu.SemaphoreType.DMA for _ in range(2 * NBUF)],
    ],
)


def _small_body(p0_ref, p1_ref, w1_ref, w2_ref, wp_ref,
                g1_ref, gt_ref, wc_ref):
    s = p0_ref[...] + p1_ref[...]
    g1_ref[...] = jnp.dot(s, w1_ref[...], preferred_element_type=jnp.float32)
    gt_ref[...] = jnp.dot(s, w2_ref[...], preferred_element_type=jnp.float32)
    wc_ref[...] = jnp.dot(w2_ref[...], wp_ref[...],
                          preferred_element_type=jnp.float32)


def _pred_body(a0_ref, a1_ref, wc_ref, pv_ref, out_ref):
    a = a0_ref[...] + a1_ref[...]
    z = jnp.dot(a, wc_ref[...], preferred_element_type=jnp.float32)
    z = z + pv_ref[0:1, :]
    mu = jnp.mean(z, axis=-1, keepdims=True)
    zc = z - mu
    var = jnp.mean(zc * zc, axis=-1, keepdims=True)
    zn = zc * lax.rsqrt(var + 1e-5) * pv_ref[1:2, :] + pv_ref[2:3, :]
    alpha = pv_ref[3, 0]
    out_ref[...] = jnp.where(zn >= 0, zn, alpha * zn)


def kernel(x, edge_index, edge_weight, batch, W1, W2, Wp, bp, gamma, beta, alpha):
    x_pad = jnp.pad(x, ((0, NPAD - N), (0, 0)))
    batch_pad = jnp.pad(batch, (0, NPAD - N))
    src = jnp.pad(edge_index[0], (0, EPAD - E))
    dst = jnp.pad(edge_index[1], (0, EPAD - E))
    ew = jnp.pad(edge_weight, (0, EPAD - E))

    agg_parts, pool_parts = _sc_call(x_pad, src, dst, ew, batch_pad)

    g1, gt, wc = pl.pallas_call(
        _small_body,
        out_shape=[
            jax.ShapeDtypeStruct((G, D), jnp.float32),
            jax.ShapeDtypeStruct((G, D), jnp.float32),
            jax.ShapeDtypeStruct((D, D), jnp.float32),
        ],
    )(pool_parts[:G], pool_parts[G:], W1, W2, Wp)

    pvec = jnp.stack([bp, gamma, beta,
                      jnp.full((D,), alpha, dtype=jnp.float32)] + [bp] * 4)

    nb = 8
    blk = NPAD // nb
    h_full = pl.pallas_call(
        _pred_body,
        grid=(nb,),
        in_specs=[
            pl.BlockSpec((blk, D), lambda i: (i, 0)),
            pl.BlockSpec((blk, D), lambda i: (i, 0)),
            pl.BlockSpec((D, D), lambda i: (0, 0)),
            pl.BlockSpec((8, D), lambda i: (0, 0)),
        ],
        out_specs=pl.BlockSpec((blk, D), lambda i: (i, 0)),
        out_shape=jax.ShapeDtypeStruct((NPAD, D), jnp.float32),
    )(agg_parts[:NPAD], agg_parts[NPAD:], wc, pvec)

    h_pred = h_full[:N]
    return (g1, g1, h_pred, h_pred, gt, gt)


# speedup vs baseline: 1.7285x; 1.3465x over previous
import functools

import jax
import jax.numpy as jnp
from jax import lax
from jax.experimental import pallas as pl
from jax.experimental.pallas import tpu as pltpu
from jax.experimental.pallas import tpu_sc as plsc

N = 10000
E = 320000
D = 128
G = 512

NC = 2
NS = 16
NW = NC * NS

NPAD = 10240
ROWS_PT = NPAD // NS
RCH = 32
NRC = ROWS_PT // RCH

ECH = 64
NBUF = 4
EPT_CH = 160
EPT = ECH * EPT_CH
EPAD = EPT * NW
POOL_PT = G // NS


def _sc_body(x_hbm, src_hbm, dst_hbm, ew_hbm, batch_hbm, xbf_hbm,
             agg_out, pool_out,
             agg_sh, pool_sh,
             src0, dst0, ew0, src1, dst1, ew1,
             src2, dst2, ew2, src3, dst3, ew3,
             rows0, rbf0, rbf1, rbf2, rbf3,
             zbuf, bidx_v,
             semg0, semg1, semg2, semg3, semi0, semi1, semi2, semi3):
    c = lax.axis_index("c")
    s = lax.axis_index("s")
    wid = s * NC + c
    r0 = s * ROWS_PT
    srcs = (src0, src1, src2, src3)
    dsts = (dst0, dst1, dst2, dst3)
    ews = (ew0, ew1, ew2, ew3)
    rows = (rows0, rows0, rows0, rows0)
    rbfs = (rbf0, rbf1, rbf2, rbf3)
    semgs = (semg0, semg1, semg2, semg3)
    semis = (semi0, semi1, semi2, semi3)

    zvec = jnp.zeros((16,), jnp.float32)

    def _zrow(i, carry):
        for j in range(D // 16):
            zbuf[i, pl.ds(j * 16, 16)] = zvec
        return carry

    lax.fori_loop(0, RCH, _zrow, 0)

    @pl.when(c == 0)
    def _():
        for k in range(NRC):
            off = r0 + k * RCH
            pltpu.sync_copy(x_hbm.at[pl.ds(off, RCH)],
                            agg_sh.at[pl.ds(off, RCH)])

    @pl.when(c != 0)
    def _():
        for k in range(NRC):
            off = r0 + k * RCH
            pltpu.sync_copy(zbuf, agg_sh.at[pl.ds(off, RCH)])

    pltpu.sync_copy(zbuf.at[pl.ds(0, POOL_PT)],
                    pool_sh.at[pl.ds(s * POOL_PT, POOL_PT)])
    plsc.subcore_barrier()

    ebase = wid * EPT

    def _scale(rows_v, ewv):
        def body(g, inner):
            wv = ewv[pl.ds(g * 16, 16)]
            for e in range(16):
                w = wv[e]
                r = g * 16 + e
                for j in range(D // 16):
                    sl = pl.ds(j * 16, 16)
                    rows_v[r, sl] = rows_v[r, sl] * w
            return inner

        lax.fori_loop(0, ECH // 16, body, 0)

    def _idx_issue(ci, q):
        e = ebase + ci * ECH
        pltpu.async_copy(src_hbm.at[pl.ds(e, ECH)], srcs[q], semis[q])
        pltpu.async_copy(dst_hbm.at[pl.ds(e, ECH)], dsts[q], semis[q])
        pltpu.async_copy(ew_hbm.at[pl.ds(e, ECH)], ews[q], semis[q])

    def _idx_wait(q):
        z = pl.ds(0, ECH)
        pltpu.make_async_copy(src_hbm.at[z], srcs[q], semis[q]).wait()
        pltpu.make_async_copy(dst_hbm.at[z], dsts[q], semis[q]).wait()
        pltpu.make_async_copy(ew_hbm.at[z], ews[q], semis[q]).wait()

    for q in range(NBUF):
        _idx_issue(q, q)
    for q in range(NBUF):
        _idx_wait(q)
        pltpu.async_copy(xbf_hbm.at[srcs[q]], rbfs[q], semgs[q])

    NIT = EPT_CH // NBUF

    def _eround(it, carry):
        for q in range(NBUF):
            ci = NBUF * it + q
            pltpu.make_async_copy(xbf_hbm.at[srcs[q]], rbfs[q],
                                  semgs[q]).wait()
            _scale(rows[q], ews[q])
            pltpu.sync_copy(rows[q], agg_sh.at[dsts[q]], add=True)

            @pl.when(ci < EPT_CH - NBUF)
            def _():
                _idx_issue(ci + NBUF, q)
                _idx_wait(q)
                pltpu.async_copy(xbf_hbm.at[srcs[q]], rbfs[q], semgs[q])

        return carry

    lax.fori_loop(0, NIT, _eround, 0)
    plsc.subcore_barrier()

    rfront = zbuf
    for k in range(NRC):
        off = r0 + k * RCH
        pltpu.sync_copy(agg_sh.at[pl.ds(off, RCH)], rfront)
        pltpu.sync_copy(batch_hbm.at[pl.ds(off, RCH)], bidx_v)
        pltpu.sync_copy(rfront, agg_out.at[pl.ds(c * NPAD + off, RCH)])
        pltpu.sync_copy(rfront, pool_sh.at[bidx_v], add=True)
    plsc.subcore_barrier()

    pltpu.sync_copy(pool_sh.at[pl.ds(s * POOL_PT, POOL_PT)],
                    pool_out.at[pl.ds(c * G + s * POOL_PT, POOL_PT)])


_sc_call = pl.kernel(
    _sc_body,
    out_type=[
        jax.ShapeDtypeStruct((NC * NPAD, D), jnp.float32),
        jax.ShapeDtypeStruct((NC * G, D), jnp.float32),
    ],
    mesh=plsc.VectorSubcoreMesh(core_axis_name="c", subcore_axis_name="s"),
    compiler_params=pltpu.CompilerParams(use_tc_tiling_on_sc=False),
    scratch_types=[
        pltpu.VMEM_SHARED((NPAD, D), jnp.float32),
        pltpu.VMEM_SHARED((G, D), jnp.float32),
        *[t for _ in range(NBUF)
          for t in (pltpu.VMEM((ECH,), jnp.int32),
                    pltpu.VMEM((ECH,), jnp.int32),
                    pltpu.VMEM((ECH,), jnp.float32))],
        pltpu.VMEM((ECH, D), jnp.float32),
        *[pltpu.VMEM((ECH, D // 2), jnp.int32) for _ in range(NBUF)],
        pltpu.VMEM((RCH, D), jnp.float32),
        pltpu.VMEM((RCH,), jnp.int32),
        *[pltpu.SemaphoreType.DMA for _ in range(2 * NBUF)],
    ],
)


def _small_body(p0_ref, p1_ref, w1_ref, w2_ref, wp_ref,
                g1_ref, gt_ref, wc_ref):
    s = p0_ref[...] + p1_ref[...]
    g1_ref[...] = jnp.dot(s, w1_ref[...], preferred_element_type=jnp.float32)
    gt_ref[...] = jnp.dot(s, w2_ref[...], preferred_element_type=jnp.float32)
    wc_ref[...] = jnp.dot(w2_ref[...], wp_ref[...],
                          preferred_element_type=jnp.float32)


def _pred_body(a0_ref, a1_ref, wc_ref, pv_ref, out_ref):
    a = a0_ref[...] + a1_ref[...]
    z = jnp.dot(a, wc_ref[...], preferred_element_type=jnp.float32)
    z = z + pv_ref[0:1, :]
    mu = jnp.mean(z, axis=-1, keepdims=True)
    zc = z - mu
    var = jnp.mean(zc * zc, axis=-1, keepdims=True)
    zn = zc * lax.rsqrt(var + 1e-5) * pv_ref[1:2, :] + pv_ref[2:3, :]
    alpha = pv_ref[3, 0]
    out_ref[...] = jnp.where(zn >= 0, zn, alpha * zn)


def kernel(x, edge_index, edge_weight, batch, W1, W2, Wp, bp, gamma, beta, alpha):
    x_pad = jnp.pad(x, ((0, NPAD - N), (0, 0)))
    batch_pad = jnp.pad(batch, (0, NPAD - N))
    src = jnp.pad(edge_index[0], (0, EPAD - E))
    dst = jnp.pad(edge_index[1], (0, EPAD - E))
    ew = jnp.pad(edge_weight, (0, EPAD - E))

    x_pk = lax.bitcast_convert_type(
        x_pad.astype(jnp.bfloat16).reshape(NPAD, D // 2, 2), jnp.int32)
    agg_parts, pool_parts = _sc_call(x_pad, src, dst, ew, batch_pad, x_pk)

    g1, gt, wc = pl.pallas_call(
        _small_body,
        out_shape=[
            jax.ShapeDtypeStruct((G, D), jnp.float32),
            jax.ShapeDtypeStruct((G, D), jnp.float32),
            jax.ShapeDtypeStruct((D, D), jnp.float32),
        ],
    )(pool_parts[:G], pool_parts[G:], W1, W2, Wp)

    pvec = jnp.stack([bp, gamma, beta,
                      jnp.full((D,), alpha, dtype=jnp.float32)] + [bp] * 4)

    nb = 8
    blk = NPAD // nb
    h_full = pl.pallas_call(
        _pred_body,
        grid=(nb,),
        in_specs=[
            pl.BlockSpec((blk, D), lambda i: (i, 0)),
            pl.BlockSpec((blk, D), lambda i: (i, 0)),
            pl.BlockSpec((D, D), lambda i: (0, 0)),
            pl.BlockSpec((8, D), lambda i: (0, 0)),
        ],
        out_specs=pl.BlockSpec((blk, D), lambda i: (i, 0)),
        out_shape=jax.ShapeDtypeStruct((NPAD, D), jnp.float32),
    )(agg_parts[:NPAD], agg_parts[NPAD:], wc, pvec)

    h_pred = h_full[:N]
    return (g1, g1, h_pred, h_pred, gt, gt)
